# Initial kernel scaffold; baseline (speedup 1.0000x reference)
import jax
import jax.numpy as jnp
from jax.experimental import pallas as pl


def kernel(x, edge_index, batch, W_in, b_in, W1, b1, W2, b2, W3, b3, W4, b4, Wd, bd, Ws, bs, Wr, br):
    raise NotImplementedError("write your pallas kernel here")


# speedup vs baseline: 10.0837x; 10.0837x over previous
import functools

import jax
import jax.numpy as jnp
from jax import lax
from jax.experimental import pallas as pl
from jax.experimental.pallas import tpu as pltpu
from jax.experimental.pallas import tpu_sc as plsc

NN = 10000
NPAD = 10240
EE = 320000
DH = 128
GG = 64
NCORE = 2
NSUB = 16
NWORK = NCORE * NSUB
EW = EE // NWORK
CH = 80
NCHUNK = EW // CH
DEGW = 16
ROWS_PER_SUB = NPAD // NSUB
BLK = 2048
NBLK = NPAD // BLK


def _mesh():
    return plsc.VectorSubcoreMesh(
        core_axis_name="c", subcore_axis_name="s", num_cores=NCORE, num_subcores=NSUB
    )


@functools.cache
def _sc_degree_kernel():
    return pl.kernel(
        _sc_degree_body,
        out_type=jax.ShapeDtypeStruct((NCORE, NPAD, DH), jnp.float32),
        mesh=_mesh(),
        scratch_types=[
            pltpu.VMEM((CH,), jnp.int32),
            pltpu.VMEM((CH, DH), jnp.float32),
            pltpu.VMEM((128, DH), jnp.float32),
            pltpu.VMEM_SHARED((NPAD, DH), jnp.float32),
        ],
    )


def _sc_degree(dst):
    return _sc_degree_kernel()(dst)[:, :, :DEGW]


def _sc_degree_body(dst_hbm, out_hbm, didx, ones_v, zbuf, acc):
    c = lax.axis_index("c")
    s = lax.axis_index("s")
    wid = s * NCORE + c

    def fill(i, _):
        for k in range(DH // 16):
            ones_v[i, pl.ds(k * 16, 16)] = jnp.ones((16,), jnp.float32)
        return 0

    lax.fori_loop(0, CH, fill, 0)

    def zrow(i, _):
        for k in range(DH // 16):
            zbuf[i, pl.ds(k * 16, 16)] = jnp.zeros((16,), jnp.float32)
        return 0

    lax.fori_loop(0, 128, zrow, 0)
    for k in range(ROWS_PER_SUB // 128):
        pltpu.sync_copy(zbuf, acc.at[pl.ds(s * ROWS_PER_SUB + k * 128, 128)])
    plsc.subcore_barrier()

    base0 = wid * EW

    def body(j, _):
        base = pl.multiple_of(base0 + j * CH, 8)
        pltpu.sync_copy(dst_hbm.at[pl.ds(base, CH)], didx)
        pltpu.sync_copy(ones_v, acc.at[didx], add=True)
        return 0

    lax.fori_loop(0, NCHUNK, body, 0)
    plsc.subcore_barrier()
    pltpu.sync_copy(
        acc.at[pl.ds(s * ROWS_PER_SUB, ROWS_PER_SUB)],
        out_hbm.at[c, pl.ds(s * ROWS_PER_SUB, ROWS_PER_SUB)],
    )


@functools.cache
def _sc_scatter_kernel():
    return pl.kernel(
        _sc_scatter_body,
        out_type=jax.ShapeDtypeStruct((NCORE, NPAD, DH), jnp.float32),
        mesh=_mesh(),
        scratch_types=[
            pltpu.VMEM((CH,), jnp.int32),
            pltpu.VMEM((CH,), jnp.int32),
            pltpu.VMEM((CH, DH), jnp.float32),
            pltpu.VMEM((128, DH), jnp.float32),
            pltpu.VMEM_SHARED((NPAD, DH), jnp.float32),
            pltpu.SemaphoreType.DMA,
        ],
    )


def _sc_scatter(h, src, dst):
    return _sc_scatter_kernel()(h, src, dst)


def _sc_scatter_body(h_hbm, src_hbm, dst_hbm, out_hbm, sidx, didx, rows, zbuf, acc, sem):
    c = lax.axis_index("c")
    s = lax.axis_index("s")
    wid = s * NCORE + c

    def zrow(i, _):
        for k in range(DH // 16):
            zbuf[i, pl.ds(k * 16, 16)] = jnp.zeros((16,), jnp.float32)
        return 0

    lax.fori_loop(0, 128, zrow, 0)
    for k in range(ROWS_PER_SUB // 128):
        pltpu.sync_copy(zbuf, acc.at[pl.ds(s * ROWS_PER_SUB + k * 128, 128)])
    plsc.subcore_barrier()

    base0 = wid * EW

    def body(j, _):
        base = pl.multiple_of(base0 + j * CH, 8)
        pltpu.sync_copy(src_hbm.at[pl.ds(base, CH)], sidx)
        pltpu.sync_copy(dst_hbm.at[pl.ds(base, CH)], didx)
        pltpu.async_copy(h_hbm.at[sidx], rows, sem).wait()
        pltpu.sync_copy(rows, acc.at[didx], add=True)
        return 0

    lax.fori_loop(0, NCHUNK, body, 0)
    plsc.subcore_barrier()
    pltpu.sync_copy(
        acc.at[pl.ds(s * ROWS_PER_SUB, ROWS_PER_SUB)],
        out_hbm.at[c, pl.ds(s * ROWS_PER_SUB, ROWS_PER_SUB)],
    )


def _dinv_from_deg(degp):
    d = degp[0, :, :1] + degp[1, :, :1] + 1.0
    return lax.rsqrt(jnp.maximum(d, 1.0))


def _tc_first_body(x_ref, win_ref, bin_ref, w1_ref, degp_ref, out_ref):
    h = jnp.maximum(
        jnp.dot(x_ref[...], win_ref[...], preferred_element_type=jnp.float32)
        + bin_ref[...],
        0.0,
    )
    hw = jnp.dot(h, w1_ref[...], preferred_element_type=jnp.float32)
    out_ref[...] = hw * _dinv_from_deg(degp_ref[...])


def _tc_mid_body(p_ref, hw_ref, degp_ref, b_ref, w_ref, out_ref):
    dinv = _dinv_from_deg(degp_ref[...])
    p = p_ref[...]
    agg = p[0] + p[1] + hw_ref[...]
    h = jnp.maximum(agg * dinv + b_ref[...], 0.0)
    out_ref[...] = jnp.dot(h, w_ref[...], preferred_element_type=jnp.float32) * dinv


def _tc_final_body(
    p_ref, hw_ref, degp_ref, b_ref, batch_ref,
    wd_ref, bd_ref, ws_ref, bs_ref, wr_ref, br_ref,
    od_ref, os_ref, or_ref, sums, cnt,
):
    i = pl.program_id(0)

    @pl.when(i == 0)
    def _init():
        sums[...] = jnp.zeros_like(sums)
        cnt[...] = jnp.zeros_like(cnt)

    dinv = _dinv_from_deg(degp_ref[...])
    p = p_ref[...]
    agg = p[0] + p[1] + hw_ref[...]
    h = jnp.maximum(agg * dinv + b_ref[...], 0.0)
    oh_t = (jnp.arange(GG, dtype=jnp.int32)[:, None] == batch_ref[...][None, :])
    oh_t = oh_t.astype(jnp.float32)
    sums[...] += jnp.dot(oh_t, h, preferred_element_type=jnp.float32)
    cnt[...] += jnp.dot(
        oh_t, jnp.ones((BLK, DH), jnp.float32), preferred_element_type=jnp.float32
    )

    @pl.when(i == pl.num_programs(0) - 1)
    def _fin():
        pooled = sums[...] / jnp.maximum(cnt[...], 1.0)
        od_ref[...] = jnp.dot(pooled, wd_ref[...], preferred_element_type=jnp.float32) + bd_ref[...]
        os_ref[...] = jnp.dot(pooled, ws_ref[...], preferred_element_type=jnp.float32) + bs_ref[...]
        or_ref[...] = jnp.dot(pooled, wr_ref[...], preferred_element_type=jnp.float32) + br_ref[...]


_row_spec = pl.BlockSpec((BLK, DH), lambda i: (i, 0))
_w_spec = pl.BlockSpec((DH, DH), lambda i: (0, 0))
_b_spec = pl.BlockSpec((1, DH), lambda i: (0, 0))
_deg_spec = pl.BlockSpec((NCORE, BLK, DEGW), lambda i: (0, i, 0))
_p_spec = pl.BlockSpec((NCORE, BLK, DH), lambda i: (0, i, 0))


def _tc_first(xp, W_in, b_in, W1, degp):
    return pl.pallas_call(
        _tc_first_body,
        grid=(NBLK,),
        in_specs=[_row_spec, _w_spec, _b_spec, _w_spec, _deg_spec],
        out_specs=_row_spec,
        out_shape=jax.ShapeDtypeStruct((NPAD, DH), jnp.float32),
    )(xp, W_in, b_in.reshape(1, DH), W1, degp)


def _tc_mid(p, hw, degp, b, W):
    return pl.pallas_call(
        _tc_mid_body,
        grid=(NBLK,),
        in_specs=[_p_spec, _row_spec, _deg_spec, _b_spec, _w_spec],
        out_specs=_row_spec,
        out_shape=jax.ShapeDtypeStruct((NPAD, DH), jnp.float32),
    )(p, hw, degp, b.reshape(1, DH), W)


def _tc_final(p, hw, degp, b4, batchp, Wd, bd, Ws, bs, Wr, br):
    g_spec = pl.BlockSpec((GG, DH), lambda i: (0, 0))
    return pl.pallas_call(
        _tc_final_body,
        grid=(NBLK,),
        in_specs=[
            _p_spec, _row_spec, _deg_spec, _b_spec,
            pl.BlockSpec((BLK,), lambda i: (i,)),
            _w_spec, _b_spec, _w_spec, _b_spec, _w_spec, _b_spec,
        ],
        out_specs=[g_spec, g_spec, g_spec],
        out_shape=[jax.ShapeDtypeStruct((GG, DH), jnp.float32)] * 3,
        scratch_shapes=[
            pltpu.VMEM((GG, DH), jnp.float32),
            pltpu.VMEM((GG, DH), jnp.float32),
        ],
    )(
        p, hw, degp, b4.reshape(1, DH), batchp,
        Wd, bd.reshape(1, DH), Ws, bs.reshape(1, DH), Wr, br.reshape(1, DH),
    )


def kernel(x, edge_index, batch, W_in, b_in, W1, b1, W2, b2, W3, b3, W4, b4,
           Wd, bd, Ws, bs, Wr, br):
    src = edge_index[0]
    dst = edge_index[1]
    xp = jnp.pad(x, ((0, NPAD - NN), (0, 0)))
    batchp = jnp.pad(batch, (0, NPAD - NN), constant_values=GG)

    degp = _sc_degree(dst)
    hw = _tc_first(xp, W_in, b_in, W1, degp)
    for b, W in ((b1, W2), (b2, W3), (b3, W4)):
        p = _sc_scatter(hw, src, dst)
        hw = _tc_mid(p, hw, degp, b, W)
    p = _sc_scatter(hw, src, dst)
    out_def, out_syn, out_rel = _tc_final(p, hw, degp, b4, batchp, Wd, bd, Ws, bs, Wr, br)
    return (out_def, out_syn, out_rel)


# speedup vs baseline: 20.2596x; 2.0092x over previous
import functools

import jax
import jax.numpy as jnp
from jax import lax
from jax.experimental import pallas as pl
from jax.experimental.pallas import tpu as pltpu
from jax.experimental.pallas import tpu_sc as plsc

NN = 10000
NPAD = 10240
EE = 320000
DH = 128
GG = 64
NCORE = 2
NSUB = 16
NWORK = NCORE * NSUB
EW = EE // NWORK
CH = 80
NCHUNK = EW // CH
DEGW = 16
ROWS_PER_SUB = NPAD // NSUB
BLK = 2048
NBLK = NPAD // BLK


def _mesh():
    return plsc.VectorSubcoreMesh(
        core_axis_name="c", subcore_axis_name="s", num_cores=NCORE, num_subcores=NSUB
    )


@functools.cache
def _sc_degree_kernel():
    return pl.kernel(
        _sc_degree_body,
        out_type=jax.ShapeDtypeStruct((NCORE, NPAD, DH), jnp.float32),
        mesh=_mesh(),
        scratch_types=[
            pltpu.VMEM((CH,), jnp.int32),
            pltpu.VMEM((CH, DH), jnp.float32),
            pltpu.VMEM((128, DH), jnp.float32),
            pltpu.VMEM_SHARED((NPAD, DH), jnp.float32),
        ],
    )


def _sc_degree(dst):
    return _sc_degree_kernel()(dst)[:, :, :DEGW]


def _sc_degree_body(dst_hbm, out_hbm, didx, ones_v, zbuf, acc):
    c = lax.axis_index("c")
    s = lax.axis_index("s")
    wid = s * NCORE + c

    def fill(i, _):
        for k in range(DH // 16):
            ones_v[i, pl.ds(k * 16, 16)] = jnp.ones((16,), jnp.float32)
        return 0

    lax.fori_loop(0, CH, fill, 0)

    def zrow(i, _):
        for k in range(DH // 16):
            zbuf[i, pl.ds(k * 16, 16)] = jnp.zeros((16,), jnp.float32)
        return 0

    lax.fori_loop(0, 128, zrow, 0)
    for k in range(ROWS_PER_SUB // 128):
        pltpu.sync_copy(zbuf, acc.at[pl.ds(s * ROWS_PER_SUB + k * 128, 128)])
    plsc.subcore_barrier()

    base0 = wid * EW

    def body(j, _):
        base = pl.multiple_of(base0 + j * CH, 8)
        pltpu.sync_copy(dst_hbm.at[pl.ds(base, CH)], didx)
        pltpu.sync_copy(ones_v, acc.at[didx], add=True)
        return 0

    lax.fori_loop(0, NCHUNK, body, 0)
    plsc.subcore_barrier()
    pltpu.sync_copy(
        acc.at[pl.ds(s * ROWS_PER_SUB, ROWS_PER_SUB)],
        out_hbm.at[c, pl.ds(s * ROWS_PER_SUB, ROWS_PER_SUB)],
    )


CB = 128
NCB = EE // CB


@functools.cache
def _sc_scatter_kernel():
    return pl.kernel(
        _sc_scatter_body,
        out_type=jax.ShapeDtypeStruct((NCORE, NPAD, DH), jnp.float32),
        mesh=_mesh(),
        scratch_types=[
            pltpu.VMEM((2, CB), jnp.int32),
            pltpu.VMEM((2, CB), jnp.int32),
            pltpu.VMEM((2, CB, DH), jnp.float32),
            pltpu.VMEM_SHARED((NPAD, DH), jnp.float32),
            pltpu.SemaphoreType.DMA,
            pltpu.SemaphoreType.DMA,
            pltpu.SemaphoreType.DMA,
            pltpu.SemaphoreType.DMA,
            pltpu.SemaphoreType.DMA,
            pltpu.SemaphoreType.DMA,
        ],
    )


def _sc_scatter(h, src, dst):
    return _sc_scatter_kernel()(h, src, dst)


def _sc_scatter_body(h_hbm, src_hbm, dst_hbm, out_hbm, sidx, didx, rows,
                     acc, isem0, isem1, gsem0, gsem1, ssem0, ssem1):
    c = lax.axis_index("c")
    s = lax.axis_index("s")
    wid = s * NCORE + c
    isem = (isem0, isem1)
    gsem = (gsem0, gsem1)
    ssem = (ssem0, ssem1)

    def zrow(i, _):
        for k in range(DH // 16):
            rows[0, i, pl.ds(k * 16, 16)] = jnp.zeros((16,), jnp.float32)
        return 0

    lax.fori_loop(0, CB, zrow, 0)
    for k in range(ROWS_PER_SUB // CB):
        pltpu.sync_copy(rows.at[0], acc.at[pl.ds(s * ROWS_PER_SUB + k * CB, CB)])
    plsc.subcore_barrier()

    nw = (NCB - wid + NWORK - 1) // NWORK

    def start_idx(k, b):
        base = pl.multiple_of((wid + k * NWORK) * CB, 8)
        pltpu.async_copy(src_hbm.at[pl.ds(base, CB)], sidx.at[b], isem[b])
        pltpu.async_copy(dst_hbm.at[pl.ds(base, CB)], didx.at[b], isem[b])

    def wait_idx(b):
        pltpu.make_async_copy(src_hbm.at[pl.ds(0, CB)], sidx.at[b], isem[b]).wait()
        pltpu.make_async_copy(dst_hbm.at[pl.ds(0, CB)], didx.at[b], isem[b]).wait()

    def start_gather(b):
        pltpu.async_copy(h_hbm.at[sidx.at[b]], rows.at[b], gsem[b])

    def wait_gather(b):
        pltpu.make_async_copy(h_hbm.at[sidx.at[b]], rows.at[b], gsem[b]).wait()

    def start_scatter(b):
        pltpu.async_copy(rows.at[b], acc.at[didx.at[b]], ssem[b], add=True)

    def wait_scatter(b):
        pltpu.make_async_copy(rows.at[b], acc.at[didx.at[b]], ssem[b]).wait()

    start_idx(0, 0)
    start_idx(1, 1)
    wait_idx(0)
    start_gather(0)
    wait_gather(0)
    start_idx(2, 0)
    start_scatter(0)

    def step(k, b):
        @pl.when(k >= 2)
        def _():
            wait_scatter(b)
        wait_idx(b)
        start_gather(b)
        wait_gather(b)

        @pl.when(k + 2 < nw)
        def _():
            start_idx(k + 2, b)
        start_scatter(b)

    def body(kk, _):
        for off, b in ((1, 1), (2, 0)):
            k = kk * 2 + off

            @pl.when(k < nw)
            def _():
                step(k, b)
        return 0

    nwmax_pairs = ((NCB + NWORK - 1) // NWORK + 1) // 2
    lax.fori_loop(0, nwmax_pairs, body, 0)
    wait_scatter(0)
    wait_scatter(1)
    plsc.subcore_barrier()
    pltpu.sync_copy(
        acc.at[pl.ds(s * ROWS_PER_SUB, ROWS_PER_SUB)],
        out_hbm.at[c, pl.ds(s * ROWS_PER_SUB, ROWS_PER_SUB)],
    )


def _dinv_from_deg(degp):
    d = degp[0, :, :1] + degp[1, :, :1] + 1.0
    return lax.rsqrt(jnp.maximum(d, 1.0))


def _tc_first_body(x_ref, win_ref, bin_ref, w1_ref, degp_ref, out_ref):
    h = jnp.maximum(
        jnp.dot(x_ref[...], win_ref[...], preferred_element_type=jnp.float32)
        + bin_ref[...],
        0.0,
    )
    hw = jnp.dot(h, w1_ref[...], preferred_element_type=jnp.float32)
    out_ref[...] = hw * _dinv_from_deg(degp_ref[...])


def _tc_mid_body(p_ref, hw_ref, degp_ref, b_ref, w_ref, out_ref):
    dinv = _dinv_from_deg(degp_ref[...])
    p = p_ref[...]
    agg = p[0] + p[1] + hw_ref[...]
    h = jnp.maximum(agg * dinv + b_ref[...], 0.0)
    out_ref[...] = jnp.dot(h, w_ref[...], preferred_element_type=jnp.float32) * dinv


def _tc_final_body(
    p_ref, hw_ref, degp_ref, b_ref, batch_ref,
    wd_ref, bd_ref, ws_ref, bs_ref, wr_ref, br_ref,
    od_ref, os_ref, or_ref, sums, cnt,
):
    i = pl.program_id(0)

    @pl.when(i == 0)
    def _init():
        sums[...] = jnp.zeros_like(sums)
        cnt[...] = jnp.zeros_like(cnt)

    dinv = _dinv_from_deg(degp_ref[...])
    p = p_ref[...]
    agg = p[0] + p[1] + hw_ref[...]
    h = jnp.maximum(agg * dinv + b_ref[...], 0.0)
    oh_t = (jnp.arange(GG, dtype=jnp.int32)[:, None] == batch_ref[...][None, :])
    oh_t = oh_t.astype(jnp.float32)
    sums[...] += jnp.dot(oh_t, h, preferred_element_type=jnp.float32)
    cnt[...] += jnp.dot(
        oh_t, jnp.ones((BLK, DH), jnp.float32), preferred_element_type=jnp.float32
    )

    @pl.when(i == pl.num_programs(0) - 1)
    def _fin():
        pooled = sums[...] / jnp.maximum(cnt[...], 1.0)
        od_ref[...] = jnp.dot(pooled, wd_ref[...], preferred_element_type=jnp.float32) + bd_ref[...]
        os_ref[...] = jnp.dot(pooled, ws_ref[...], preferred_element_type=jnp.float32) + bs_ref[...]
        or_ref[...] = jnp.dot(pooled, wr_ref[...], preferred_element_type=jnp.float32) + br_ref[...]


_row_spec = pl.BlockSpec((BLK, DH), lambda i: (i, 0))
_w_spec = pl.BlockSpec((DH, DH), lambda i: (0, 0))
_b_spec = pl.BlockSpec((1, DH), lambda i: (0, 0))
_deg_spec = pl.BlockSpec((NCORE, BLK, DEGW), lambda i: (0, i, 0))
_p_spec = pl.BlockSpec((NCORE, BLK, DH), lambda i: (0, i, 0))


def _tc_first(xp, W_in, b_in, W1, degp):
    return pl.pallas_call(
        _tc_first_body,
        grid=(NBLK,),
        in_specs=[_row_spec, _w_spec, _b_spec, _w_spec, _deg_spec],
        out_specs=_row_spec,
        out_shape=jax.ShapeDtypeStruct((NPAD, DH), jnp.float32),
    )(xp, W_in, b_in.reshape(1, DH), W1, degp)


def _tc_mid(p, hw, degp, b, W):
    return pl.pallas_call(
        _tc_mid_body,
        grid=(NBLK,),
        in_specs=[_p_spec, _row_spec, _deg_spec, _b_spec, _w_spec],
        out_specs=_row_spec,
        out_shape=jax.ShapeDtypeStruct((NPAD, DH), jnp.float32),
    )(p, hw, degp, b.reshape(1, DH), W)


def _tc_final(p, hw, degp, b4, batchp, Wd, bd, Ws, bs, Wr, br):
    g_spec = pl.BlockSpec((GG, DH), lambda i: (0, 0))
    return pl.pallas_call(
        _tc_final_body,
        grid=(NBLK,),
        in_specs=[
            _p_spec, _row_spec, _deg_spec, _b_spec,
            pl.BlockSpec((BLK,), lambda i: (i,)),
            _w_spec, _b_spec, _w_spec, _b_spec, _w_spec, _b_spec,
        ],
        out_specs=[g_spec, g_spec, g_spec],
        out_shape=[jax.ShapeDtypeStruct((GG, DH), jnp.float32)] * 3,
        scratch_shapes=[
            pltpu.VMEM((GG, DH), jnp.float32),
            pltpu.VMEM((GG, DH), jnp.float32),
        ],
    )(
        p, hw, degp, b4.reshape(1, DH), batchp,
        Wd, bd.reshape(1, DH), Ws, bs.reshape(1, DH), Wr, br.reshape(1, DH),
    )


def kernel(x, edge_index, batch, W_in, b_in, W1, b1, W2, b2, W3, b3, W4, b4,
           Wd, bd, Ws, bs, Wr, br):
    src = edge_index[0]
    dst = edge_index[1]
    xp = jnp.pad(x, ((0, NPAD - NN), (0, 0)))
    batchp = jnp.pad(batch, (0, NPAD - NN), constant_values=GG)

    degp = _sc_degree(dst)
    hw = _tc_first(xp, W_in, b_in, W1, degp)
    for b, W in ((b1, W2), (b2, W3), (b3, W4)):
        p = _sc_scatter(hw, src, dst)
        hw = _tc_mid(p, hw, degp, b, W)
    p = _sc_scatter(hw, src, dst)
    out_def, out_syn, out_rel = _tc_final(p, hw, degp, b4, batchp, Wd, bd, Ws, bs, Wr, br)
    return (out_def, out_syn, out_rel)
